# baseline (speedup 1.0000x reference)
import math

import jax
import jax.numpy as jnp
from jax import lax
from jax.experimental import pallas as pl
from jax.experimental.pallas import tpu as pltpu
from jax.experimental.pallas import tpu_sc as plsc

B, S, D, F, E, TOPK = 2, 2048, 1024, 1024, 64, 1
CAP = int(math.ceil(B * S * TOPK * 1.0 / E))
T = B * S
NSLOT = E * CAP + CAP
BT = 512
NBG = T // BT

_SC_INFO = plsc.get_sparse_core_info()
NC, NS = _SC_INFO.num_cores, _SC_INFO.num_subcores
NW = NC * NS
TPW = T // NW
RCH = 32
NCH = TPW // RCH
LPR = D // 16


EP = 128


def _gate_body(x_ref, wg_ref, dst_ref, g_ref, se_ref, xbf_ref, counts_ref):
    pid = pl.program_id(0)

    @pl.when(pid == 0)
    def _():
        counts_ref[...] = jnp.zeros_like(counts_ref)

    xv = x_ref[...]
    xbf_ref[...] = xv.astype(jnp.bfloat16)
    l = jnp.dot(xv, wg_ref[...], preferred_element_type=jnp.float32)
    eio = lax.broadcasted_iota(jnp.int32, (BT, EP), 1)
    l = jnp.where(eio < E, l, -3e38)
    m = jnp.max(l, axis=1, keepdims=True)
    idx = jnp.min(jnp.where(l == m, eio, EP), axis=1)
    score = 1.0 / jnp.sum(jnp.exp(l - m), axis=1)

    oh = (eio == idx[:, None]).astype(jnp.float32)
    ti = lax.broadcasted_iota(jnp.int32, (BT, BT), 0)
    tj = lax.broadcasted_iota(jnp.int32, (BT, BT), 1)
    tril = (tj <= ti).astype(jnp.float32)
    csum = jnp.dot(tril, oh, preferred_element_type=jnp.float32)
    posw = csum - oh
    counts = counts_ref[0, :]
    pos = jnp.sum(oh * (counts[None, :] + posw), axis=1).astype(jnp.int32)
    counts_ref[0, :] = counts + jnp.sum(oh, axis=0)

    keep = pos < CAP
    slot = idx * CAP + jnp.where(keep, pos, 0)
    dst_ref[...] = jnp.where(keep, idx * CAP + pos, E * CAP).reshape(1, 1, BT)
    g_ref[...] = slot.reshape(1, 1, BT)
    se_ref[...] = jnp.where(keep, score, 0.0).reshape(1, 1, BT)


def _gate(xf, Wg):
    Wgp = jnp.concatenate(
        [Wg, jnp.zeros((D, EP - E), dtype=Wg.dtype)], axis=1)
    dst, g, se, xbf = pl.pallas_call(
        _gate_body,
        grid=(NBG,),
        in_specs=[
            pl.BlockSpec((BT, D), lambda i: (i, 0)),
            pl.BlockSpec((D, EP), lambda i: (0, 0)),
        ],
        out_specs=[
            pl.BlockSpec((1, 1, BT), lambda i: (i, 0, 0)),
            pl.BlockSpec((1, 1, BT), lambda i: (i, 0, 0)),
            pl.BlockSpec((1, 1, BT), lambda i: (i, 0, 0)),
            pl.BlockSpec((BT, D), lambda i: (i, 0)),
        ],
        out_shape=[
            jax.ShapeDtypeStruct((NBG, 1, BT), jnp.int32),
            jax.ShapeDtypeStruct((NBG, 1, BT), jnp.int32),
            jax.ShapeDtypeStruct((NBG, 1, BT), jnp.float32),
            jax.ShapeDtypeStruct((T, D), jnp.bfloat16),
        ],
        scratch_shapes=[pltpu.VMEM((1, EP), jnp.float32)],
    )(xf, Wgp)
    return dst.reshape(T), g.reshape(T), se.reshape(T), xbf


def _dispatch_body(xf_hbm, dst3_hbm, disp_hbm, rows0, rows1, idx_v, seml, sems):
    wid = lax.axis_index("s") * NC + lax.axis_index("c")
    bufs = (rows0, rows1)
    pltpu.sync_copy(dst3_hbm.at[wid], idx_v)

    loads = [None] * NCH
    scats = [None] * NCH
    loads[0] = pltpu.async_copy(
        xf_hbm.at[pl.ds(wid * TPW, RCH)], bufs[0], seml)
    for c in range(NCH):
        b = c & 1
        if c + 1 < NCH:
            if c >= 1:
                scats[c - 1].wait()
            loads[c + 1] = pltpu.async_copy(
                xf_hbm.at[pl.ds(wid * TPW + (c + 1) * RCH, RCH)],
                bufs[1 - b], seml)
        loads[c].wait()
        scats[c] = pltpu.async_copy(bufs[b], disp_hbm.at[idx_v.at[c]], sems)
    scats[NCH - 2].wait()
    scats[NCH - 1].wait()


def _dispatch(xf, dst3):
    mesh = plsc.VectorSubcoreMesh(core_axis_name="c", subcore_axis_name="s")
    return pl.kernel(
        _dispatch_body,
        out_type=jax.ShapeDtypeStruct((NSLOT, D // 2), jnp.int32),
        mesh=mesh,
        scratch_types=[
            pltpu.VMEM((RCH, D // 2), jnp.int32),
            pltpu.VMEM((RCH, D // 2), jnp.int32),
            pltpu.VMEM((NCH, RCH), jnp.int32),
            pltpu.SemaphoreType.DMA,
            pltpu.SemaphoreType.DMA,
        ],
    )(xf, dst3)


def _ffn_body(disp_ref, w1_ref, b1_ref, w2_ref, b2_ref, o_ref):
    x = disp_ref[...]
    h = jnp.dot(x, w1_ref[0].astype(jnp.bfloat16),
                preferred_element_type=jnp.float32) + b1_ref[0]
    h = jnp.maximum(h, 0.0).astype(jnp.bfloat16)
    o_ref[...] = jnp.dot(h, w2_ref[0].astype(jnp.bfloat16),
                         preferred_element_type=jnp.float32) + b2_ref[0]


def _ffn(disp, W1, b1, W2, b2):
    wmap = lambda e: (e, 0, 0)
    return pl.pallas_call(
        _ffn_body,
        grid=(E,),
        in_specs=[
            pl.BlockSpec((CAP, D), lambda e: (e, 0)),

            pl.BlockSpec((1, D, F), wmap),
            pl.BlockSpec((1, 1, F), wmap),
            pl.BlockSpec((1, F, D), wmap),
            pl.BlockSpec((1, 1, D), wmap),
        ],
        out_specs=pl.BlockSpec((CAP, D), lambda e: (e, 0)),
        out_shape=jax.ShapeDtypeStruct((E * CAP, D), jnp.float32),
    )(disp, W1, b1.reshape(E, 1, F), W2, b2.reshape(E, 1, D))


def _combine_body(o_hbm, g3_hbm, se3_hbm, y_hbm, rows0, rows1, idx_v, se_v,
                  semg, semst):
    wid = lax.axis_index("s") * NC + lax.axis_index("c")
    bufs = (rows0, rows1)
    pltpu.sync_copy(g3_hbm.at[wid], idx_v)
    pltpu.sync_copy(se3_hbm.at[wid], se_v)

    gaths = [None] * NCH
    stores = [None] * NCH
    gaths[0] = pltpu.async_copy(o_hbm.at[idx_v.at[0]], bufs[0], semg)

    for c in range(NCH):
        b = c & 1
        if c + 1 < NCH:
            if c >= 1:
                stores[c - 1].wait()
            gaths[c + 1] = pltpu.async_copy(
                o_hbm.at[idx_v.at[c + 1]], bufs[1 - b], semg)
        gaths[c].wait()
        rows = bufs[b]

        def scale_grp(gi, _, c=c, rows=rows):
            sg = se_v[c, pl.ds(gi * 16, 16)]
            for j in range(16):
                s = sg[j]
                r = gi * 16 + j
                for v in range(LPR):
                    sl = pl.ds(v * 16, 16)
                    rows[r, sl] = rows[r, sl] * s
            return 0

        lax.fori_loop(0, RCH // 16, scale_grp, 0)
        stores[c] = pltpu.async_copy(
            rows, y_hbm.at[pl.ds(wid * TPW + c * RCH, RCH)], semst)
    stores[NCH - 2].wait()
    stores[NCH - 1].wait()


def _combine(o, g3, se3):
    mesh = plsc.VectorSubcoreMesh(core_axis_name="c", subcore_axis_name="s")
    return pl.kernel(
        _combine_body,
        out_type=jax.ShapeDtypeStruct((T, D), jnp.float32),
        mesh=mesh,
        scratch_types=[
            pltpu.VMEM((RCH, D), jnp.float32),
            pltpu.VMEM((RCH, D), jnp.float32),
            pltpu.VMEM((NCH, RCH), jnp.int32),
            pltpu.VMEM((NCH, RCH), jnp.float32),
            pltpu.SemaphoreType.DMA,
            pltpu.SemaphoreType.DMA,
        ],
    )(o, g3, se3)


def kernel(x, Wg, W1, b1, W2, b2):
    xf = x.reshape(T, D)
    dst, g, se, xbf = _gate(xf, Wg)
    xi = lax.bitcast_convert_type(
        xbf.reshape(T, D // 2, 2), jnp.int32)
    disp_i = _dispatch(xi, dst.reshape(NW, NCH, RCH))
    disp = lax.bitcast_convert_type(disp_i, jnp.bfloat16).reshape(NSLOT, D)
    o = _ffn(disp, W1, b1, W2, b2)
    y = _combine(o, g.reshape(NW, NCH, RCH), se.reshape(NW, NCH, RCH))
    return y.reshape(B, S, D)


# speedup vs baseline: 1.7179x; 1.7179x over previous
import math

import jax
import jax.numpy as jnp
from jax import lax
from jax.experimental import pallas as pl
from jax.experimental.pallas import tpu as pltpu
from jax.experimental.pallas import tpu_sc as plsc

B, S, D, F, E, TOPK = 2, 2048, 1024, 1024, 64, 1
CAP = int(math.ceil(B * S * TOPK * 1.0 / E))
T = B * S
NSLOT = E * CAP + CAP
BT = 512
NBG = T // BT

_SC_INFO = plsc.get_sparse_core_info()
NC, NS = _SC_INFO.num_cores, _SC_INFO.num_subcores
NW = NC * NS
TPW = T // NW
RCH = 32
NCH = TPW // RCH
LPR = D // 16


EP = 128


def _gate_body(x_ref, wg_ref, dst_ref, g_ref, se_ref, counts_ref):
    pid = pl.program_id(0)

    @pl.when(pid == 0)
    def _():
        counts_ref[...] = jnp.zeros_like(counts_ref)

    l = jnp.dot(x_ref[...], wg_ref[...], preferred_element_type=jnp.float32)
    eio = lax.broadcasted_iota(jnp.int32, (BT, EP), 1)
    l = jnp.where(eio < E, l, -3e38)
    m = jnp.max(l, axis=1, keepdims=True)
    idx = jnp.min(jnp.where(l == m, eio, EP), axis=1)
    score = 1.0 / jnp.sum(jnp.exp(l - m), axis=1)

    oh = (eio == idx[:, None]).astype(jnp.float32)
    ti = lax.broadcasted_iota(jnp.int32, (BT, BT), 0)
    tj = lax.broadcasted_iota(jnp.int32, (BT, BT), 1)
    tril = (tj <= ti).astype(jnp.float32)
    csum = jnp.dot(tril, oh, preferred_element_type=jnp.float32)
    posw = csum - oh
    counts = counts_ref[0, :]
    pos = jnp.sum(oh * (counts[None, :] + posw), axis=1).astype(jnp.int32)
    counts_ref[0, :] = counts + jnp.sum(oh, axis=0)

    keep = pos < CAP
    slot = idx * CAP + jnp.where(keep, pos, 0)
    dst_ref[...] = jnp.where(keep, idx * CAP + pos, E * CAP).reshape(1, 1, BT)
    g_ref[...] = slot.reshape(1, 1, BT)
    se_ref[...] = jnp.where(keep, score, 0.0).reshape(1, 1, BT)


def _gate(xf, Wg):
    Wgp = jnp.concatenate(
        [Wg, jnp.zeros((D, EP - E), dtype=Wg.dtype)], axis=1)
    dst, g, se = pl.pallas_call(
        _gate_body,
        grid=(NBG,),
        in_specs=[
            pl.BlockSpec((BT, D), lambda i: (i, 0)),
            pl.BlockSpec((D, EP), lambda i: (0, 0)),
        ],
        out_specs=[
            pl.BlockSpec((1, 1, BT), lambda i: (i, 0, 0)),
            pl.BlockSpec((1, 1, BT), lambda i: (i, 0, 0)),
            pl.BlockSpec((1, 1, BT), lambda i: (i, 0, 0)),
        ],
        out_shape=[
            jax.ShapeDtypeStruct((NBG, 1, BT), jnp.int32),
            jax.ShapeDtypeStruct((NBG, 1, BT), jnp.int32),
            jax.ShapeDtypeStruct((NBG, 1, BT), jnp.float32),
        ],
        scratch_shapes=[pltpu.VMEM((1, EP), jnp.float32)],
    )(xf, Wgp)
    return dst.reshape(T), g.reshape(T), se.reshape(T)


def _dispatch_body(xf_hbm, dst3_hbm, disp_hbm, rows0, rows1, idx_v, seml, sems):
    wid = lax.axis_index("s") * NC + lax.axis_index("c")
    bufs = (rows0, rows1)
    pltpu.sync_copy(dst3_hbm.at[wid], idx_v)

    loads = [None] * NCH
    scats = [None] * NCH
    loads[0] = pltpu.async_copy(
        xf_hbm.at[pl.ds(wid * TPW, RCH)], bufs[0], seml)
    for c in range(NCH):
        b = c & 1
        if c + 1 < NCH:
            if c >= 1:
                scats[c - 1].wait()
            loads[c + 1] = pltpu.async_copy(
                xf_hbm.at[pl.ds(wid * TPW + (c + 1) * RCH, RCH)],
                bufs[1 - b], seml)
        loads[c].wait()
        scats[c] = pltpu.async_copy(bufs[b], disp_hbm.at[idx_v.at[c]], sems)
    scats[NCH - 2].wait()
    scats[NCH - 1].wait()


def _dispatch(xf, dst3):
    mesh = plsc.VectorSubcoreMesh(core_axis_name="c", subcore_axis_name="s")
    return pl.kernel(
        _dispatch_body,
        out_type=jax.ShapeDtypeStruct((NSLOT, D), jnp.float32),
        mesh=mesh,
        scratch_types=[
            pltpu.VMEM((RCH, D), jnp.float32),
            pltpu.VMEM((RCH, D), jnp.float32),
            pltpu.VMEM((NCH, RCH), jnp.int32),
            pltpu.SemaphoreType.DMA,
            pltpu.SemaphoreType.DMA,
        ],
    )(xf, dst3)


def _ffn_body(disp_ref, w1_ref, b1_ref, w2_ref, b2_ref, o_ref):
    x = disp_ref[...].astype(jnp.bfloat16)
    h = jnp.dot(x, w1_ref[0].astype(jnp.bfloat16),
                preferred_element_type=jnp.float32) + b1_ref[0]
    h = jnp.maximum(h, 0.0).astype(jnp.bfloat16)
    o_ref[...] = jnp.dot(h, w2_ref[0].astype(jnp.bfloat16),
                         preferred_element_type=jnp.float32) + b2_ref[0]


def _ffn(disp, W1, b1, W2, b2):
    wmap = lambda e: (e, 0, 0)
    return pl.pallas_call(
        _ffn_body,
        grid=(E,),
        in_specs=[
            pl.BlockSpec((CAP, D), lambda e: (e, 0)),

            pl.BlockSpec((1, D, F), wmap),
            pl.BlockSpec((1, 1, F), wmap),
            pl.BlockSpec((1, F, D), wmap),
            pl.BlockSpec((1, 1, D), wmap),
        ],
        out_specs=pl.BlockSpec((CAP, D), lambda e: (e, 0)),
        out_shape=jax.ShapeDtypeStruct((E * CAP, D), jnp.float32),
    )(disp, W1, b1.reshape(E, 1, F), W2, b2.reshape(E, 1, D))


def _combine_body(o_hbm, g3_hbm, se3_hbm, y_hbm, rows0, rows1, idx_v, se_v,
                  semg, semst):
    wid = lax.axis_index("s") * NC + lax.axis_index("c")
    bufs = (rows0, rows1)
    pltpu.sync_copy(g3_hbm.at[wid], idx_v)
    pltpu.sync_copy(se3_hbm.at[wid], se_v)

    gaths = [None] * NCH
    stores = [None] * NCH
    gaths[0] = pltpu.async_copy(o_hbm.at[idx_v.at[0]], bufs[0], semg)

    for c in range(NCH):
        b = c & 1
        if c + 1 < NCH:
            if c >= 1:
                stores[c - 1].wait()
            gaths[c + 1] = pltpu.async_copy(
                o_hbm.at[idx_v.at[c + 1]], bufs[1 - b], semg)
        gaths[c].wait()
        rows = bufs[b]

        def scale_grp(gi, _, c=c, rows=rows):
            sg = se_v[c, pl.ds(gi * 16, 16)]
            for j in range(16):
                s = sg[j]
                r = gi * 16 + j
                for v in range(LPR):
                    sl = pl.ds(v * 16, 16)
                    rows[r, sl] = rows[r, sl] * s
            return 0

        lax.fori_loop(0, RCH // 16, scale_grp, 0)
        stores[c] = pltpu.async_copy(
            rows, y_hbm.at[pl.ds(wid * TPW + c * RCH, RCH)], semst)
    stores[NCH - 2].wait()
    stores[NCH - 1].wait()


def _combine(o, g3, se3):
    mesh = plsc.VectorSubcoreMesh(core_axis_name="c", subcore_axis_name="s")
    return pl.kernel(
        _combine_body,
        out_type=jax.ShapeDtypeStruct((T, D), jnp.float32),
        mesh=mesh,
        scratch_types=[
            pltpu.VMEM((RCH, D), jnp.float32),
            pltpu.VMEM((RCH, D), jnp.float32),
            pltpu.VMEM((NCH, RCH), jnp.int32),
            pltpu.VMEM((NCH, RCH), jnp.float32),
            pltpu.SemaphoreType.DMA,
            pltpu.SemaphoreType.DMA,
        ],
    )(o, g3, se3)


def kernel(x, Wg, W1, b1, W2, b2):
    xf = x.reshape(T, D)
    dst, g, se = _gate(xf, Wg)
    disp = _dispatch(xf, dst.reshape(NW, NCH, RCH))
    o = _ffn(disp, W1, b1, W2, b2)
    y = _combine(o, g.reshape(NW, NCH, RCH), se.reshape(NW, NCH, RCH))
    return y.reshape(B, S, D)
